# Initial kernel scaffold; baseline (speedup 1.0000x reference)
import jax
import jax.numpy as jnp
from jax.experimental import pallas as pl


def kernel(x, edge_index, W1, att_src1, att_dst1, b1, W2, att_src2, att_dst2, b2):
    raise NotImplementedError("write your pallas kernel here")


# speedup vs baseline: 1.0134x; 1.0134x over previous
import functools

import jax
import jax.numpy as jnp
from jax.experimental import pallas as pl
from jax.experimental.pallas import tpu as pltpu


def _mm_kernel(x_ref, w_ref, o_ref):
    o_ref[...] = jnp.dot(x_ref[...], w_ref[...],
                         preferred_element_type=jnp.float32)


def _matmul(x, wT):
    n, k = x.shape
    m = wT.shape[1]
    bn = 512
    npad = -n % bn
    xp = jnp.pad(x, ((0, npad), (0, 0)))
    out = pl.pallas_call(
        _mm_kernel,
        grid=((n + npad) // bn,),
        in_specs=[pl.BlockSpec((bn, k), lambda i: (i, 0)),
                  pl.BlockSpec((k, m), lambda i: (0, 0))],
        out_specs=pl.BlockSpec((bn, m), lambda i: (i, 0)),
        out_shape=jax.ShapeDtypeStruct((n + npad, m), jnp.float32),
    )(xp, wT)
    return out[:n]


def _gat_layer(x, src, dst, W, a_src, a_dst, b, heads, out_dim):
    n = x.shape[0]
    h = _matmul(x, W.T).reshape(n, heads, out_dim)
    alpha_src = jnp.sum(h * a_src[None, :, :], axis=-1)
    alpha_dst = jnp.sum(h * a_dst[None, :, :], axis=-1)
    alpha = alpha_src[src] + alpha_dst[dst]
    alpha = jax.nn.leaky_relu(alpha, negative_slope=0.2)
    amax = jax.ops.segment_max(alpha, dst, num_segments=n)
    ea = jnp.exp(alpha - amax[dst])
    denom = jax.ops.segment_sum(ea, dst, num_segments=n)
    coef = ea / (denom[dst] + 1e-16)
    msg = h[src] * coef[:, :, None]
    out = jax.ops.segment_sum(msg, dst, num_segments=n)
    return out.reshape(n, heads * out_dim) + b


def kernel(x, edge_index, W1, att_src1, att_dst1, b1,
           W2, att_src2, att_dst2, b2):
    n = x.shape[0]
    loops = jnp.arange(n, dtype=edge_index.dtype)
    src = jnp.concatenate([edge_index[0], loops])
    dst = jnp.concatenate([edge_index[1], loops])
    h1 = _gat_layer(x, src, dst, W1, att_src1, att_dst1, b1,
                    heads=8, out_dim=8)
    h2 = _gat_layer(h1, src, dst, W2, att_src2, att_dst2, b2,
                    heads=1, out_dim=x.shape[1])
    return h2


# speedup vs baseline: 14.5131x; 14.3213x over previous
import functools

import jax
import jax.numpy as jnp
from jax import lax
from jax.experimental import pallas as pl
from jax.experimental.pallas import tpu as pltpu
from jax.experimental.pallas import tpu_sc as plsc

NT = 32
NSUB = 16
K = 128
BN = 400
F32 = jnp.float32


def _splat(v, dtype=jnp.int32):
    return jnp.full((16,), v, dtype)


def _dyn_gather(x, idx):
    return lax.gather(
        x, idx[:, None],
        lax.GatherDimensionNumbers(offset_dims=(), collapsed_slice_dims=(0,),
                                   start_index_map=(0,)),
        (1,), mode=lax.GatherScatterMode.PROMISE_IN_BOUNDS)


def _dot(a, b):
    return jnp.dot(a, b, preferred_element_type=F32)


def _make_prep_kernel(nparts, mid):
    def kern(*refs):
        if mid:
            p_ref, b_ref, w_ref, asb_ref, adb_ref = refs[:5]
            outs = refs[5:]
            o = p_ref[0] + p_ref[1] + b_ref[...]
        else:
            o_ref, w_ref, asb_ref, adb_ref = refs[:4]
            outs = refs[4:]
            o = o_ref[...]
        h = _dot(o, w_ref[...])
        for i in range(nparts):
            outs[i][...] = h[:, 64 * i:64 * (i + 1)]
        outs[nparts][...] = _dot(h, asb_ref[...])
        outs[nparts + 1][...] = _dot(h, adb_ref[...])
    return kern


def _amax_kernel(mp_ref, ad_ref, r_ref):
    m = jnp.max(mp_ref[...], axis=0)
    ad = ad_ref[...]
    am = ad + m
    am = jnp.where(am >= 0, am, 0.2 * am)
    r_ref[...] = jnp.concatenate([ad, am], axis=1)


def _denom_kernel(dp_ref, t_ref):
    t = 1.0 / (jnp.sum(dp_ref[...], axis=0) + 1e-16)
    t_ref[...] = jnp.concatenate([t, t], axis=1)


def _fin_kernel(pa_ref, pb_ref, b_ref, o_ref):
    o_ref[...] = jnp.concatenate([pa_ref[0] + pa_ref[1],
                                  pb_ref[0] + pb_ref[1]], axis=1) + b_ref[...]


def _tc_prep(x, w, asb, adb, mid, b=None):
    din, dh = w.shape
    nparts = dh // 64
    n = x.shape[1] if mid else x.shape[0]
    grid = (n // BN,)
    kern = _make_prep_kernel(nparts, mid)
    ins = [x]
    in_specs = [pl.BlockSpec(((2, BN, din) if mid else (BN, din)),
                             ((lambda i: (0, i, 0)) if mid else (lambda i: (i, 0))))]
    if mid:
        ins.append(b)
        in_specs.append(pl.BlockSpec((1, din), lambda i: (0, 0)))
    ins += [w, asb, adb]
    in_specs += [pl.BlockSpec((din, dh), lambda i: (0, 0)),
                 pl.BlockSpec((dh, 16), lambda i: (0, 0)),
                 pl.BlockSpec((dh, 8), lambda i: (0, 0))]
    out = pl.pallas_call(
        kern,
        grid=grid,
        in_specs=in_specs,
        out_specs=[pl.BlockSpec((BN, 64), lambda i: (i, 0))] * nparts +
                  [pl.BlockSpec((BN, 16), lambda i: (i, 0)),
                   pl.BlockSpec((BN, 8), lambda i: (i, 0))],
        out_shape=[jax.ShapeDtypeStruct((n, 64), F32)] * nparts +
                  [jax.ShapeDtypeStruct((n, 16), F32),
                   jax.ShapeDtypeStruct((n, 8), F32)],
    )(*ins)
    return out[:nparts], out[nparts], out[nparts + 1]


def _tc_amax(mp, ad):
    n = ad.shape[0]
    return pl.pallas_call(
        _amax_kernel,
        grid=(n // BN,),
        in_specs=[pl.BlockSpec((NT, BN, 8), lambda i: (0, i, 0)),
                  pl.BlockSpec((BN, 8), lambda i: (i, 0))],
        out_specs=pl.BlockSpec((BN, 16), lambda i: (i, 0)),
        out_shape=jax.ShapeDtypeStruct((n, 16), F32),
    )(mp, ad)


def _tc_denom(dp):
    n = dp.shape[1]
    return pl.pallas_call(
        _denom_kernel,
        grid=(n // BN,),
        in_specs=[pl.BlockSpec((NT, BN, 8), lambda i: (0, i, 0))],
        out_specs=pl.BlockSpec((BN, 16), lambda i: (i, 0)),
        out_shape=jax.ShapeDtypeStruct((n, 16), F32),
    )(dp)


def _tc_fin(pa, pb, b):
    _, n, dp = pa.shape
    d = 2 * dp
    return pl.pallas_call(
        _fin_kernel,
        grid=(n // BN,),
        in_specs=[pl.BlockSpec((2, BN, dp), lambda i: (0, i, 0)),
                  pl.BlockSpec((2, BN, dp), lambda i: (0, i, 0)),
                  pl.BlockSpec((1, d), lambda i: (0, 0))],
        out_specs=pl.BlockSpec((BN, d), lambda i: (i, 0)),
        out_shape=jax.ShapeDtypeStruct((n, d), F32),
    )(pa, pb, b)


def _wid():
    return lax.axis_index("s") * 2 + lax.axis_index("c")


@functools.cache
def _sc_phase_a(n, ch, ep_real):
    mesh = plsc.VectorSubcoreMesh(core_axis_name="c", subcore_axis_name="s")

    @functools.partial(
        pl.kernel,
        out_type=jax.ShapeDtypeStruct((NT, n * 8), F32),
        mesh=mesh,
        compiler_params=pltpu.CompilerParams(needs_layout_passes=False, use_tc_tiling_on_sc=False),
        scratch_types=[
            pltpu.VMEM((ch, K), jnp.int32),
            pltpu.VMEM((ch, K), jnp.int32),
            pltpu.VMEM((K, 16), F32),
            pltpu.VMEM((n * 8,), F32),
            pltpu.SemaphoreType.DMA,
        ],
    )
    def body(src_hbm, dst_hbm, s_hbm, mp_hbm, srcbuf, dstbuf, srows, m, sem):
        w = _wid()
        iota = lax.iota(jnp.int32, 16)
        col = lax.bitwise_and(iota, 7)
        mask8 = iota < 8

        def initb(i, c):
            m[pl.ds(i * 16, 16)] = jnp.full((16,), -1e30, F32)
            return c
        lax.fori_loop(0, n * 8 // 16, initb, 0)

        pltpu.sync_copy(src_hbm.at[w], srcbuf)
        pltpu.sync_copy(dst_hbm.at[w], dstbuf)

        def chunk(j, c):
            pltpu.async_copy(s_hbm.at[srcbuf.at[j]], srows, sem).wait()
            base = (w * ch + j) * K

            def edge(e, c2):
                dv = plsc.load_gather(dstbuf, [_splat(j), _splat(e)])
                aval = srows[e]
                midx = dv * 8 + col
                mk = mask8 & (_splat(base + e) < ep_real)
                cur = plsc.load_gather(m, [midx], mask=mk)
                plsc.store_scatter(m, [midx], jnp.maximum(cur, aval), mask=mk)
                return c2
            lax.fori_loop(0, K, edge, 0)
            return c
        lax.fori_loop(0, ch, chunk, 0)
        pltpu.sync_copy(m, mp_hbm.at[w])

    return body


@functools.cache
def _sc_phase_b(n, ch, ep_real):
    mesh = plsc.VectorSubcoreMesh(core_axis_name="c", subcore_axis_name="s")

    @functools.partial(
        pl.kernel,
        out_type=[jax.ShapeDtypeStruct((NT * ch, K * 8), F32),
                  jax.ShapeDtypeStruct((NT, n * 8), F32)],
        mesh=mesh,
        compiler_params=pltpu.CompilerParams(needs_layout_passes=False, use_tc_tiling_on_sc=False),
        scratch_types=[
            pltpu.VMEM((ch, K), jnp.int32),
            pltpu.VMEM((ch, K), jnp.int32),
            pltpu.VMEM((K, 16), F32),
            pltpu.VMEM((K, 16), F32),
            pltpu.VMEM((K * 8,), F32),
            pltpu.VMEM((n * 8,), F32),
            pltpu.SemaphoreType.DMA,
            pltpu.SemaphoreType.DMA,
        ],
    )
    def body(src_hbm, dst_hbm, s_hbm, r_hbm, ea_hbm, dp_hbm,
             srcbuf, dstbuf, srows, rrows, eabuf, dnm, sem, sem2):
        w = _wid()
        iota = lax.iota(jnp.int32, 16)
        col = lax.bitwise_and(iota, 7)
        mask8 = iota < 8

        def initb(i, c):
            dnm[pl.ds(i * 16, 16)] = jnp.zeros((16,), F32)
            return c
        lax.fori_loop(0, n * 8 // 16, initb, 0)

        pltpu.sync_copy(src_hbm.at[w], srcbuf)
        pltpu.sync_copy(dst_hbm.at[w], dstbuf)

        def chunk(j, c):
            pltpu.async_copy(s_hbm.at[srcbuf.at[j]], srows, sem).wait()
            pltpu.async_copy(r_hbm.at[dstbuf.at[j]], rrows, sem2).wait()
            base = (w * ch + j) * K

            def edge(e, c2):
                sv = srows[e]
                rv = rrows[e]
                rlo = _dyn_gather(rv, col)
                rhi = _dyn_gather(rv, col + 8)
                alpha = sv + rlo
                alpha = jnp.where(alpha >= 0, alpha, 0.2 * alpha)
                ea = jnp.exp(alpha - rhi)
                ea = jnp.where(_splat(base + e) < ep_real, ea,
                               jnp.zeros((16,), F32))
                plsc.store_scatter(eabuf, [e * 8 + col], ea, mask=mask8)
                dv = plsc.load_gather(dstbuf, [_splat(j), _splat(e)])
                plsc.addupdate_scatter(dnm, [dv * 8 + col], ea, mask=mask8)
                return c2
            lax.fori_loop(0, K, edge, 0)
            pltpu.sync_copy(eabuf, ea_hbm.at[w * ch + j])
            return c
        lax.fori_loop(0, ch, chunk, 0)
        pltpu.sync_copy(dnm, dp_hbm.at[w])

    return body


@functools.cache
def _sc_phase_c(n, ch, d, heads):
    nv = d // 16
    rows_per_tile = n // NSUB
    zrows = 5
    mesh = plsc.VectorSubcoreMesh(core_axis_name="c", subcore_axis_name="s")

    @functools.partial(
        pl.kernel,
        out_type=jax.ShapeDtypeStruct((2, n, d), F32),
        mesh=mesh,
        compiler_params=pltpu.CompilerParams(needs_layout_passes=False, use_tc_tiling_on_sc=False),
        scratch_types=[
            pltpu.VMEM((ch, K), jnp.int32),
            pltpu.VMEM((ch, K), jnp.int32),
            pltpu.VMEM((K, 16), F32),
            pltpu.VMEM((K * 8,), F32),
            pltpu.VMEM((K, d), F32),
            pltpu.VMEM((K, d), F32),
            pltpu.VMEM((zrows, d), F32),
            pltpu.VMEM_SHARED((n, d), F32),
            pltpu.SemaphoreType.DMA,
            pltpu.SemaphoreType.DMA,
        ],
    )
    def body(src_hbm, dst_hbm, t_hbm, ea_hbm, h_hbm, op_hbm,
             srcbuf, dstbuf, trows, eabuf, hrows, msgbuf, zbuf, acc, sem, sem2):
        cid = lax.axis_index("c")
        sid = lax.axis_index("s")
        w = _wid()
        iota = lax.iota(jnp.int32, 16)

        zero = jnp.zeros((16,), F32)
        for zi in range(zrows):
            for zv in range(nv):
                zbuf[zi, pl.ds(16 * zv, 16)] = zero

        def zinit(i, c):
            pltpu.sync_copy(zbuf, acc.at[pl.ds(sid * rows_per_tile + i * zrows,
                                               zrows)])
            return c
        lax.fori_loop(0, rows_per_tile // zrows, zinit, 0)
        plsc.subcore_barrier()

        pltpu.sync_copy(src_hbm.at[w], srcbuf)
        pltpu.sync_copy(dst_hbm.at[w], dstbuf)

        if heads == 1:
            pats = [jnp.zeros((16,), jnp.int32)] * nv
        else:
            pats = [2 * v + (iota >= 8).astype(jnp.int32) for v in range(nv)]

        def chunk(j, c):
            pltpu.async_copy(t_hbm.at[dstbuf.at[j]], trows, sem).wait()
            pltpu.async_copy(h_hbm.at[srcbuf.at[j]], hrows, sem2).wait()
            pltpu.sync_copy(ea_hbm.at[w * ch + j], eabuf)

            def edge(e, c2):
                tv = trows[e]
                for v in range(nv):
                    eav = plsc.load_gather(eabuf, [e * 8 + pats[v]])
                    tvv = _dyn_gather(tv, pats[v])
                    hv = hrows[e, pl.ds(16 * v, 16)]
                    msgbuf[e, pl.ds(16 * v, 16)] = eav * tvv * hv
                return c2
            lax.fori_loop(0, K, edge, 0)
            pltpu.sync_copy(msgbuf, acc.at[dstbuf.at[j]], add=True)
            return c
        lax.fori_loop(0, ch, chunk, 0)
        plsc.subcore_barrier()

        pltpu.sync_copy(acc.at[pl.ds(sid * rows_per_tile, rows_per_tile)],
                        op_hbm.at[cid, pl.ds(sid * rows_per_tile,
                                             rows_per_tile)])

    return body


def _block_diag_att(att, dup):
    heads, hd = att.shape
    dh = heads * hd
    rows = jnp.arange(dh)
    cols = jnp.repeat(jnp.arange(heads), hd)
    m = jnp.zeros((dh, heads), F32).at[rows, cols].set(att.reshape(dh))
    if heads == 1:
        m = jnp.tile(m, (1, 8))
    if dup:
        m = jnp.concatenate([m, m], axis=1)
    return m


def _gat_layer(src3, dst3, n, ch, ep_real, x, W, att_src, att_dst,
               heads, mid, bias_in=None):
    asb = _block_diag_att(att_src, dup=True)
    adb = _block_diag_att(att_dst, dup=False)
    wT = W.T
    if mid:
        hs, s, ad = _tc_prep(x, wT, asb, adb, mid=True,
                             b=bias_in.reshape(1, -1))
    else:
        hs, s, ad = _tc_prep(x, wT, asb, adb, mid=False)
    mp = _sc_phase_a(n, ch, ep_real)(src3, dst3, s)
    r = _tc_amax(mp.reshape(NT, n, 8), ad)
    ea, dp = _sc_phase_b(n, ch, ep_real)(src3, dst3, s, r)
    t = _tc_denom(dp.reshape(NT, n, 8))
    phase_c = _sc_phase_c(n, ch, 64, heads)
    return [phase_c(src3, dst3, t, ea, hp) for hp in hs]


def kernel(x, edge_index, W1, att_src1, att_dst1, b1,
           W2, att_src2, att_dst2, b2):
    n = x.shape[0]
    e = edge_index.shape[1]
    ep_real = e + n
    ch = -(-ep_real // (NT * K))
    epp = NT * ch * K
    loops = jnp.arange(n, dtype=edge_index.dtype)
    pad = jnp.zeros((epp - ep_real,), edge_index.dtype)
    src1d = jnp.concatenate([edge_index[0], loops, pad])
    dst1d = jnp.concatenate([edge_index[1], loops, pad])
    src3 = src1d.reshape(NT, ch, K)
    dst3 = dst1d.reshape(NT, ch, K)

    op1 = _gat_layer(src3, dst3, n, ch, ep_real, x, W1, att_src1,
                     att_dst1, heads=8, mid=False)
    op2 = _gat_layer(src3, dst3, n, ch, ep_real, op1[0], W2, att_src2,
                     att_dst2, heads=1, mid=True, bias_in=b1)
    return _tc_fin(op2[0], op2[1], b2.reshape(1, -1))


# speedup vs baseline: 20.1800x; 1.3905x over previous
import functools

import jax
import jax.numpy as jnp
from jax import lax
from jax.experimental import pallas as pl
from jax.experimental.pallas import tpu as pltpu
from jax.experimental.pallas import tpu_sc as plsc

NT = 32
NSUB = 16
K = 128
BN = 400
F32 = jnp.float32


def _splat(v, dtype=jnp.int32):
    return jnp.full((16,), v, dtype)


def _dyn_gather(x, idx):
    return lax.gather(
        x, idx[:, None],
        lax.GatherDimensionNumbers(offset_dims=(), collapsed_slice_dims=(0,),
                                   start_index_map=(0,)),
        (1,), mode=lax.GatherScatterMode.PROMISE_IN_BOUNDS)


def _dot(a, b):
    return jnp.dot(a, b, preferred_element_type=F32)


def _make_prep_kernel(nparts, mid):
    def kern(*refs):
        if mid:
            p_ref, b_ref, w_ref, asb_ref, adb_ref = refs[:5]
            outs = refs[5:]
            o = p_ref[0] + p_ref[1] + b_ref[...]
        else:
            o_ref, w_ref, asb_ref, adb_ref = refs[:4]
            outs = refs[4:]
            o = o_ref[...]
        h = _dot(o, w_ref[...])
        for i in range(nparts):
            outs[i][...] = h[:, 64 * i:64 * (i + 1)]
        outs[nparts][...] = _dot(h, asb_ref[...])
        outs[nparts + 1][...] = _dot(h, adb_ref[...])
    return kern


def _amax_kernel(mp_ref, ad_ref, r_ref):
    m = jnp.max(mp_ref[...], axis=0)
    ad = ad_ref[...]
    am = ad + m
    am = jnp.where(am >= 0, am, 0.2 * am)
    r_ref[...] = jnp.concatenate([ad, am], axis=1)


def _denom_kernel(dp_ref, t_ref):
    t = 1.0 / (jnp.sum(dp_ref[...], axis=0) + 1e-16)
    t_ref[...] = jnp.concatenate([t, t], axis=1)


def _fin_kernel(pa_ref, pb_ref, b_ref, o_ref):
    o_ref[...] = jnp.concatenate([pa_ref[0] + pa_ref[1],
                                  pb_ref[0] + pb_ref[1]], axis=1) + b_ref[...]


def _tc_prep(x, w, asb, adb, mid, b=None):
    din, dh = w.shape
    nparts = dh // 64
    n = x.shape[1] if mid else x.shape[0]
    grid = (n // BN,)
    kern = _make_prep_kernel(nparts, mid)
    ins = [x]
    in_specs = [pl.BlockSpec(((2, BN, din) if mid else (BN, din)),
                             ((lambda i: (0, i, 0)) if mid else (lambda i: (i, 0))))]
    if mid:
        ins.append(b)
        in_specs.append(pl.BlockSpec((1, din), lambda i: (0, 0)))
    ins += [w, asb, adb]
    in_specs += [pl.BlockSpec((din, dh), lambda i: (0, 0)),
                 pl.BlockSpec((dh, 16), lambda i: (0, 0)),
                 pl.BlockSpec((dh, 8), lambda i: (0, 0))]
    out = pl.pallas_call(
        kern,
        grid=grid,
        in_specs=in_specs,
        out_specs=[pl.BlockSpec((BN, 64), lambda i: (i, 0))] * nparts +
                  [pl.BlockSpec((BN, 16), lambda i: (i, 0)),
                   pl.BlockSpec((BN, 8), lambda i: (i, 0))],
        out_shape=[jax.ShapeDtypeStruct((n, 64), F32)] * nparts +
                  [jax.ShapeDtypeStruct((n, 16), F32),
                   jax.ShapeDtypeStruct((n, 8), F32)],
    )(*ins)
    return out[:nparts], out[nparts], out[nparts + 1]


def _tc_amax(mp, ad):
    n = ad.shape[0]
    return pl.pallas_call(
        _amax_kernel,
        grid=(n // BN,),
        in_specs=[pl.BlockSpec((NT, BN, 8), lambda i: (0, i, 0)),
                  pl.BlockSpec((BN, 8), lambda i: (i, 0))],
        out_specs=pl.BlockSpec((BN, 16), lambda i: (i, 0)),
        out_shape=jax.ShapeDtypeStruct((n, 16), F32),
    )(mp, ad)


def _tc_denom(dp):
    n = dp.shape[1]
    return pl.pallas_call(
        _denom_kernel,
        grid=(n // BN,),
        in_specs=[pl.BlockSpec((NT, BN, 8), lambda i: (0, i, 0))],
        out_specs=pl.BlockSpec((BN, 16), lambda i: (i, 0)),
        out_shape=jax.ShapeDtypeStruct((n, 16), F32),
    )(dp)


def _tc_fin(pa, pb, b):
    _, n, dp = pa.shape
    d = 2 * dp
    return pl.pallas_call(
        _fin_kernel,
        grid=(n // BN,),
        in_specs=[pl.BlockSpec((2, BN, dp), lambda i: (0, i, 0)),
                  pl.BlockSpec((2, BN, dp), lambda i: (0, i, 0)),
                  pl.BlockSpec((1, d), lambda i: (0, 0))],
        out_specs=pl.BlockSpec((BN, d), lambda i: (i, 0)),
        out_shape=jax.ShapeDtypeStruct((n, d), F32),
    )(pa, pb, b)


def _wid():
    return lax.axis_index("s") * 2 + lax.axis_index("c")


@functools.cache
def _sc_phase_a(n, ch, ep_real):
    mesh = plsc.VectorSubcoreMesh(core_axis_name="c", subcore_axis_name="s")

    @functools.partial(
        pl.kernel,
        out_type=jax.ShapeDtypeStruct((NT, n * 8), F32),
        mesh=mesh,
        compiler_params=pltpu.CompilerParams(needs_layout_passes=False, use_tc_tiling_on_sc=False),
        scratch_types=[
            pltpu.VMEM((ch, K), jnp.int32),
            pltpu.VMEM((ch, K), jnp.int32),
            pltpu.VMEM((K, 16), F32),
            pltpu.VMEM((n * 8,), F32),
            pltpu.SemaphoreType.DMA,
        ],
    )
    def body(src_hbm, dst_hbm, s_hbm, mp_hbm, srcbuf, dstbuf, srows, m, sem):
        w = _wid()
        iota = lax.iota(jnp.int32, 16)
        col = lax.bitwise_and(iota, 7)
        mask8 = iota < 8

        def initb(i, c):
            m[pl.ds(i * 16, 16)] = jnp.full((16,), -1e30, F32)
            return c
        lax.fori_loop(0, n * 8 // 16, initb, 0)

        pltpu.sync_copy(src_hbm.at[w], srcbuf)
        pltpu.sync_copy(dst_hbm.at[w], dstbuf)

        def chunk(j, c):
            pltpu.async_copy(s_hbm.at[srcbuf.at[j]], srows, sem).wait()
            base = (w * ch + j) * K

            def edge(e, c2):
                dv = plsc.load_gather(dstbuf, [_splat(j), _splat(e)])
                aval = srows[e]
                midx = dv * 8 + col
                mk = mask8 & (_splat(base + e) < ep_real)
                cur = plsc.load_gather(m, [midx], mask=mk)
                plsc.store_scatter(m, [midx], jnp.maximum(cur, aval), mask=mk)
                return c2
            lax.fori_loop(0, K, edge, 0)
            return c
        lax.fori_loop(0, ch, chunk, 0)
        pltpu.sync_copy(m, mp_hbm.at[w])

    return body


@functools.cache
def _sc_phase_b(n, ch, ep_real):
    mesh = plsc.VectorSubcoreMesh(core_axis_name="c", subcore_axis_name="s")

    @functools.partial(
        pl.kernel,
        out_type=[jax.ShapeDtypeStruct((NT * ch, K * 8), F32),
                  jax.ShapeDtypeStruct((NT, n * 8), F32)],
        mesh=mesh,
        compiler_params=pltpu.CompilerParams(needs_layout_passes=False, use_tc_tiling_on_sc=False),
        scratch_types=[
            pltpu.VMEM((ch, K), jnp.int32),
            pltpu.VMEM((ch, K), jnp.int32),
            pltpu.VMEM((K, 16), F32),
            pltpu.VMEM((K, 16), F32),
            pltpu.VMEM((K * 8,), F32),
            pltpu.VMEM((n * 8,), F32),
            pltpu.SemaphoreType.DMA,
            pltpu.SemaphoreType.DMA,
        ],
    )
    def body(src_hbm, dst_hbm, s_hbm, r_hbm, ea_hbm, dp_hbm,
             srcbuf, dstbuf, srows, rrows, eabuf, dnm, sem, sem2):
        w = _wid()
        iota = lax.iota(jnp.int32, 16)
        col = lax.bitwise_and(iota, 7)
        mask8 = iota < 8

        def initb(i, c):
            dnm[pl.ds(i * 16, 16)] = jnp.zeros((16,), F32)
            return c
        lax.fori_loop(0, n * 8 // 16, initb, 0)

        pltpu.sync_copy(src_hbm.at[w], srcbuf)
        pltpu.sync_copy(dst_hbm.at[w], dstbuf)

        def chunk(j, c):
            pltpu.async_copy(s_hbm.at[srcbuf.at[j]], srows, sem).wait()
            pltpu.async_copy(r_hbm.at[dstbuf.at[j]], rrows, sem2).wait()
            base = (w * ch + j) * K

            @plsc.parallel_loop(0, K)
            def edge(e):
                sv = srows[e]
                rv = rrows[e]
                rlo = _dyn_gather(rv, col)
                rhi = _dyn_gather(rv, col + 8)
                alpha = sv + rlo
                alpha = jnp.where(alpha >= 0, alpha, 0.2 * alpha)
                ea = jnp.exp(alpha - rhi)
                ea = jnp.where(_splat(base + e) < ep_real, ea,
                               jnp.zeros((16,), F32))
                plsc.store_scatter(eabuf, [e * 8 + col], ea, mask=mask8)
                dv = plsc.load_gather(dstbuf, [_splat(j), _splat(e)])
                plsc.addupdate_scatter(dnm, [dv * 8 + col], ea, mask=mask8)
            pltpu.sync_copy(eabuf, ea_hbm.at[w * ch + j])
            return c
        lax.fori_loop(0, ch, chunk, 0)
        pltpu.sync_copy(dnm, dp_hbm.at[w])

    return body


@functools.cache
def _sc_phase_c(n, ch, d, heads):
    nv = d // 16
    rows_per_tile = n // NSUB
    zrows = 5
    mesh = plsc.VectorSubcoreMesh(core_axis_name="c", subcore_axis_name="s")

    @functools.partial(
        pl.kernel,
        out_type=jax.ShapeDtypeStruct((2, n, d), F32),
        mesh=mesh,
        compiler_params=pltpu.CompilerParams(needs_layout_passes=False, use_tc_tiling_on_sc=False),
        scratch_types=[
            pltpu.VMEM((ch, K), jnp.int32),
            pltpu.VMEM((ch, K), jnp.int32),
            pltpu.VMEM((K, 16), F32),
            pltpu.VMEM((K * 8,), F32),
            pltpu.VMEM((K, d), F32),
            pltpu.VMEM((K, d), F32),
            pltpu.VMEM((zrows, d), F32),
            pltpu.VMEM_SHARED((n, d), F32),
            pltpu.SemaphoreType.DMA,
            pltpu.SemaphoreType.DMA,
        ],
    )
    def body(src_hbm, dst_hbm, t_hbm, ea_hbm, h_hbm, op_hbm,
             srcbuf, dstbuf, trows, eabuf, hrows, msgbuf, zbuf, acc, sem, sem2):
        cid = lax.axis_index("c")
        sid = lax.axis_index("s")
        w = _wid()
        iota = lax.iota(jnp.int32, 16)

        zero = jnp.zeros((16,), F32)
        for zi in range(zrows):
            for zv in range(nv):
                zbuf[zi, pl.ds(16 * zv, 16)] = zero

        def zinit(i, c):
            pltpu.sync_copy(zbuf, acc.at[pl.ds(sid * rows_per_tile + i * zrows,
                                               zrows)])
            return c
        lax.fori_loop(0, rows_per_tile // zrows, zinit, 0)
        plsc.subcore_barrier()

        pltpu.sync_copy(src_hbm.at[w], srcbuf)
        pltpu.sync_copy(dst_hbm.at[w], dstbuf)

        if heads == 1:
            pats = [jnp.zeros((16,), jnp.int32)] * nv
        else:
            pats = [2 * v + (iota >= 8).astype(jnp.int32) for v in range(nv)]

        def chunk(j, c):
            pltpu.async_copy(t_hbm.at[dstbuf.at[j]], trows, sem).wait()
            pltpu.async_copy(h_hbm.at[srcbuf.at[j]], hrows, sem2).wait()
            pltpu.sync_copy(ea_hbm.at[w * ch + j], eabuf)

            @plsc.parallel_loop(0, K)
            def edge(e):
                tv = trows[e]
                for v in range(nv):
                    eav = plsc.load_gather(eabuf, [e * 8 + pats[v]])
                    tvv = _dyn_gather(tv, pats[v])
                    hv = hrows[e, pl.ds(16 * v, 16)]
                    msgbuf[e, pl.ds(16 * v, 16)] = eav * tvv * hv
            pltpu.sync_copy(msgbuf, acc.at[dstbuf.at[j]], add=True)
            return c
        lax.fori_loop(0, ch, chunk, 0)
        plsc.subcore_barrier()

        pltpu.sync_copy(acc.at[pl.ds(sid * rows_per_tile, rows_per_tile)],
                        op_hbm.at[cid, pl.ds(sid * rows_per_tile,
                                             rows_per_tile)])

    return body


def _block_diag_att(att, dup):
    heads, hd = att.shape
    dh = heads * hd
    rows = jnp.arange(dh)
    cols = jnp.repeat(jnp.arange(heads), hd)
    m = jnp.zeros((dh, heads), F32).at[rows, cols].set(att.reshape(dh))
    if heads == 1:
        m = jnp.tile(m, (1, 8))
    if dup:
        m = jnp.concatenate([m, m], axis=1)
    return m


def _gat_layer(src3, dst3, n, ch, ep_real, x, W, att_src, att_dst,
               heads, mid, bias_in=None):
    asb = _block_diag_att(att_src, dup=True)
    adb = _block_diag_att(att_dst, dup=False)
    wT = W.T
    if mid:
        hs, s, ad = _tc_prep(x, wT, asb, adb, mid=True,
                             b=bias_in.reshape(1, -1))
    else:
        hs, s, ad = _tc_prep(x, wT, asb, adb, mid=False)
    mp = _sc_phase_a(n, ch, ep_real)(src3, dst3, s)
    r = _tc_amax(mp.reshape(NT, n, 8), ad)
    ea, dp = _sc_phase_b(n, ch, ep_real)(src3, dst3, s, r)
    t = _tc_denom(dp.reshape(NT, n, 8))
    phase_c = _sc_phase_c(n, ch, 64, heads)
    return [phase_c(src3, dst3, t, ea, hp) for hp in hs]


def kernel(x, edge_index, W1, att_src1, att_dst1, b1,
           W2, att_src2, att_dst2, b2):
    n = x.shape[0]
    e = edge_index.shape[1]
    ep_real = e + n
    ch = -(-ep_real // (NT * K))
    epp = NT * ch * K
    loops = jnp.arange(n, dtype=edge_index.dtype)
    pad = jnp.zeros((epp - ep_real,), edge_index.dtype)
    src1d = jnp.concatenate([edge_index[0], loops, pad])
    dst1d = jnp.concatenate([edge_index[1], loops, pad])
    src3 = src1d.reshape(NT, ch, K)
    dst3 = dst1d.reshape(NT, ch, K)

    op1 = _gat_layer(src3, dst3, n, ch, ep_real, x, W1, att_src1,
                     att_dst1, heads=8, mid=False)
    op2 = _gat_layer(src3, dst3, n, ch, ep_real, op1[0], W2, att_src2,
                     att_dst2, heads=1, mid=True, bias_in=b1)
    return _tc_fin(op2[0], op2[1], b2.reshape(1, -1))


# speedup vs baseline: 21.5047x; 1.0656x over previous
import functools

import jax
import jax.numpy as jnp
from jax import lax
from jax.experimental import pallas as pl
from jax.experimental.pallas import tpu as pltpu
from jax.experimental.pallas import tpu_sc as plsc

NT = 32
NSUB = 16
K = 128
BN = 400
F32 = jnp.float32


def _splat(v, dtype=jnp.int32):
    return jnp.full((16,), v, dtype)


def _dyn_gather(x, idx):
    return lax.gather(
        x, idx[:, None],
        lax.GatherDimensionNumbers(offset_dims=(), collapsed_slice_dims=(0,),
                                   start_index_map=(0,)),
        (1,), mode=lax.GatherScatterMode.PROMISE_IN_BOUNDS)


def _dot(a, b):
    return jnp.dot(a, b, preferred_element_type=F32)


def _make_prep_kernel(nparts, mid):
    def kern(*refs):
        if mid:
            p_ref, b_ref, w_ref, asb_ref, adb_ref = refs[:5]
            outs = refs[5:]
            o = p_ref[0] + p_ref[1] + b_ref[...]
        else:
            o_ref, w_ref, asb_ref, adb_ref = refs[:4]
            outs = refs[4:]
            o = o_ref[...]
        h = _dot(o, w_ref[...])
        for i in range(nparts):
            outs[i][...] = h[:, 64 * i:64 * (i + 1)]
        outs[nparts][...] = _dot(h, asb_ref[...])
        outs[nparts + 1][...] = _dot(h, adb_ref[...])
    return kern


def _amax_kernel(mp_ref, ad_ref, r_ref):
    m = jnp.max(mp_ref[...], axis=0)
    ad = ad_ref[...]
    am = ad + m
    am = jnp.where(am >= 0, am, 0.2 * am)
    r_ref[...] = jnp.concatenate([ad, am], axis=1)


def _denom_kernel(dp_ref, t_ref):
    t = 1.0 / (jnp.sum(dp_ref[...], axis=0) + 1e-16)
    t_ref[...] = jnp.concatenate([t, t], axis=1)


def _fin_kernel(pa_ref, pb_ref, b_ref, o_ref):
    o_ref[...] = jnp.concatenate([pa_ref[0] + pa_ref[1],
                                  pb_ref[0] + pb_ref[1]], axis=1) + b_ref[...]


def _tc_prep(x, w, asb, adb, mid, b=None):
    din, dh = w.shape
    nparts = dh // 64
    n = x.shape[1] if mid else x.shape[0]
    grid = (n // BN,)
    kern = _make_prep_kernel(nparts, mid)
    ins = [x]
    in_specs = [pl.BlockSpec(((2, BN, din) if mid else (BN, din)),
                             ((lambda i: (0, i, 0)) if mid else (lambda i: (i, 0))))]
    if mid:
        ins.append(b)
        in_specs.append(pl.BlockSpec((1, din), lambda i: (0, 0)))
    ins += [w, asb, adb]
    in_specs += [pl.BlockSpec((din, dh), lambda i: (0, 0)),
                 pl.BlockSpec((dh, 16), lambda i: (0, 0)),
                 pl.BlockSpec((dh, 8), lambda i: (0, 0))]
    out = pl.pallas_call(
        kern,
        grid=grid,
        in_specs=in_specs,
        out_specs=[pl.BlockSpec((BN, 64), lambda i: (i, 0))] * nparts +
                  [pl.BlockSpec((BN, 16), lambda i: (i, 0)),
                   pl.BlockSpec((BN, 8), lambda i: (i, 0))],
        out_shape=[jax.ShapeDtypeStruct((n, 64), F32)] * nparts +
                  [jax.ShapeDtypeStruct((n, 16), F32),
                   jax.ShapeDtypeStruct((n, 8), F32)],
    )(*ins)
    return out[:nparts], out[nparts], out[nparts + 1]


def _tc_amax(mp, ad):
    n = ad.shape[0]
    return pl.pallas_call(
        _amax_kernel,
        grid=(n // BN,),
        in_specs=[pl.BlockSpec((NT, BN, 8), lambda i: (0, i, 0)),
                  pl.BlockSpec((BN, 8), lambda i: (i, 0))],
        out_specs=pl.BlockSpec((BN, 16), lambda i: (i, 0)),
        out_shape=jax.ShapeDtypeStruct((n, 16), F32),
    )(mp, ad)


def _tc_denom(dp):
    n = dp.shape[1]
    return pl.pallas_call(
        _denom_kernel,
        grid=(n // BN,),
        in_specs=[pl.BlockSpec((NT, BN, 8), lambda i: (0, i, 0))],
        out_specs=pl.BlockSpec((BN, 16), lambda i: (i, 0)),
        out_shape=jax.ShapeDtypeStruct((n, 16), F32),
    )(dp)


def _tc_fin(pa, pb, b):
    _, n, dp = pa.shape
    d = 2 * dp
    return pl.pallas_call(
        _fin_kernel,
        grid=(n // BN,),
        in_specs=[pl.BlockSpec((2, BN, dp), lambda i: (0, i, 0)),
                  pl.BlockSpec((2, BN, dp), lambda i: (0, i, 0)),
                  pl.BlockSpec((1, d), lambda i: (0, 0))],
        out_specs=pl.BlockSpec((BN, d), lambda i: (i, 0)),
        out_shape=jax.ShapeDtypeStruct((n, d), F32),
    )(pa, pb, b)


def _wid():
    return lax.axis_index("s") * 2 + lax.axis_index("c")


@functools.cache
def _sc_phase_a(n, ch, ep_real):
    mesh = plsc.VectorSubcoreMesh(core_axis_name="c", subcore_axis_name="s")

    @functools.partial(
        pl.kernel,
        out_type=jax.ShapeDtypeStruct((NT, n * 8), F32),
        mesh=mesh,
        compiler_params=pltpu.CompilerParams(needs_layout_passes=False, use_tc_tiling_on_sc=False),
        scratch_types=[
            pltpu.VMEM((ch, K), jnp.int32),
            pltpu.VMEM((ch, K), jnp.int32),
            pltpu.VMEM((K, 16), F32),
            pltpu.VMEM((n * 8,), F32),
            pltpu.SemaphoreType.DMA,
        ],
    )
    def body(src_hbm, dst_hbm, s_hbm, mp_hbm, srcbuf, dstbuf, srows, m, sem):
        w = _wid()
        iota = lax.iota(jnp.int32, 16)
        col = lax.bitwise_and(iota, 7)
        mask8 = iota < 8

        def initb(i, c):
            m[pl.ds(i * 16, 16)] = jnp.full((16,), -1e30, F32)
            return c
        lax.fori_loop(0, n * 8 // 16, initb, 0)

        pltpu.sync_copy(src_hbm.at[w], srcbuf)
        pltpu.sync_copy(dst_hbm.at[w], dstbuf)

        swapidx = lax.bitwise_xor(iota, 8)

        def chunk(j, c):
            pltpu.async_copy(s_hbm.at[srcbuf.at[j]], srows, sem).wait()

            def edge(q, c2):
                e0 = 2 * q
                e1 = 2 * q + 1
                comb = jnp.where(mask8, srows[e0], srows[e1])
                dv0 = plsc.load_gather(dstbuf, [_splat(j), _splat(e0)])
                dv1 = plsc.load_gather(dstbuf, [_splat(j), _splat(e1)])
                midx = jnp.where(mask8, dv0, dv1) * 8 + col
                eqd = dv0 == dv1
                swapped = _dyn_gather(comb, swapidx)
                val = jnp.where(eqd, jnp.maximum(comb, swapped), comb)
                mk = mask8 | (dv0 != dv1)
                cur = plsc.load_gather(m, [midx], mask=mk)
                plsc.store_scatter(m, [midx], jnp.maximum(cur, val), mask=mk)
                return c2
            lax.fori_loop(0, K // 2, edge, 0)
            return c
        lax.fori_loop(0, ch, chunk, 0)
        pltpu.sync_copy(m, mp_hbm.at[w])

    return body


@functools.cache
def _sc_phase_b(n, ch, ep_real):
    mesh = plsc.VectorSubcoreMesh(core_axis_name="c", subcore_axis_name="s")

    @functools.partial(
        pl.kernel,
        out_type=[jax.ShapeDtypeStruct((NT * ch, K * 8), F32),
                  jax.ShapeDtypeStruct((NT, n * 8), F32)],
        mesh=mesh,
        compiler_params=pltpu.CompilerParams(needs_layout_passes=False, use_tc_tiling_on_sc=False),
        scratch_types=[
            pltpu.VMEM((ch, K), jnp.int32),
            pltpu.VMEM((ch, K), jnp.int32),
            pltpu.VMEM((K, 16), F32),
            pltpu.VMEM((K, 16), F32),
            pltpu.VMEM((K * 8,), F32),
            pltpu.VMEM((n * 8,), F32),
            pltpu.SemaphoreType.DMA,
            pltpu.SemaphoreType.DMA,
        ],
    )
    def body(src_hbm, dst_hbm, s_hbm, r_hbm, ea_hbm, dp_hbm,
             srcbuf, dstbuf, srows, rrows, eabuf, dnm, sem, sem2):
        w = _wid()
        iota = lax.iota(jnp.int32, 16)
        col = lax.bitwise_and(iota, 7)
        mask8 = iota < 8

        def initb(i, c):
            dnm[pl.ds(i * 16, 16)] = jnp.zeros((16,), F32)
            return c
        lax.fori_loop(0, n * 8 // 16, initb, 0)

        pltpu.sync_copy(src_hbm.at[w], srcbuf)
        pltpu.sync_copy(dst_hbm.at[w], dstbuf)

        def chunk(j, c):
            pltpu.async_copy(s_hbm.at[srcbuf.at[j]], srows, sem).wait()
            pltpu.async_copy(r_hbm.at[dstbuf.at[j]], rrows, sem2).wait()
            base = (w * ch + j) * K

            @plsc.parallel_loop(0, K)
            def edge(e):
                sv = srows[e]
                rv = rrows[e]
                rlo = _dyn_gather(rv, col)
                rhi = _dyn_gather(rv, col + 8)
                alpha = sv + rlo
                alpha = jnp.where(alpha >= 0, alpha, 0.2 * alpha)
                ea = jnp.exp(alpha - rhi)
                ea = jnp.where(_splat(base + e) < ep_real, ea,
                               jnp.zeros((16,), F32))
                plsc.store_scatter(eabuf, [e * 8 + col], ea, mask=mask8)
                dv = plsc.load_gather(dstbuf, [_splat(j), _splat(e)])
                plsc.addupdate_scatter(dnm, [dv * 8 + col], ea, mask=mask8)
            pltpu.sync_copy(eabuf, ea_hbm.at[w * ch + j])
            return c
        lax.fori_loop(0, ch, chunk, 0)
        pltpu.sync_copy(dnm, dp_hbm.at[w])

    return body


@functools.cache
def _sc_phase_c(n, ch, d, heads):
    nv = d // 16
    rows_per_tile = n // NSUB
    zrows = 5
    mesh = plsc.VectorSubcoreMesh(core_axis_name="c", subcore_axis_name="s")

    @functools.partial(
        pl.kernel,
        out_type=jax.ShapeDtypeStruct((2, n, d), F32),
        mesh=mesh,
        compiler_params=pltpu.CompilerParams(needs_layout_passes=False, use_tc_tiling_on_sc=False),
        scratch_types=[
            pltpu.VMEM((ch, K), jnp.int32),
            pltpu.VMEM((ch, K), jnp.int32),
            pltpu.VMEM((K, 16), F32),
            pltpu.VMEM((K * 8,), F32),
            pltpu.VMEM((K, d), F32),
            pltpu.VMEM((K, d), F32),
            pltpu.VMEM((zrows, d), F32),
            pltpu.VMEM_SHARED((n, d), F32),
            pltpu.SemaphoreType.DMA,
            pltpu.SemaphoreType.DMA,
        ],
    )
    def body(src_hbm, dst_hbm, t_hbm, ea_hbm, h_hbm, op_hbm,
             srcbuf, dstbuf, trows, eabuf, hrows, msgbuf, zbuf, acc, sem, sem2):
        cid = lax.axis_index("c")
        sid = lax.axis_index("s")
        w = _wid()
        iota = lax.iota(jnp.int32, 16)
        col = lax.bitwise_and(iota, 7)

        zero = jnp.zeros((16,), F32)
        for zi in range(zrows):
            for zv in range(nv):
                zbuf[zi, pl.ds(16 * zv, 16)] = zero

        def zinit(i, c):
            pltpu.sync_copy(zbuf, acc.at[pl.ds(sid * rows_per_tile + i * zrows,
                                               zrows)])
            return c
        lax.fori_loop(0, rows_per_tile // zrows, zinit, 0)
        plsc.subcore_barrier()

        pltpu.sync_copy(src_hbm.at[w], srcbuf)
        pltpu.sync_copy(dst_hbm.at[w], dstbuf)

        if heads == 1:
            pats = [jnp.zeros((16,), jnp.int32)] * nv
        else:
            pats = [2 * v + (iota >= 8).astype(jnp.int32) for v in range(nv)]

        def chunk(j, c):
            pltpu.async_copy(t_hbm.at[dstbuf.at[j]], trows, sem).wait()
            pltpu.async_copy(h_hbm.at[srcbuf.at[j]], hrows, sem2).wait()
            pltpu.sync_copy(ea_hbm.at[w * ch + j], eabuf)

            @plsc.parallel_loop(0, K)
            def edge(e):
                ea16 = plsc.load_gather(eabuf, [e * 8 + col])
                coef = ea16 * trows[e]
                for v in range(nv):
                    cexp = _dyn_gather(coef, pats[v])
                    hv = hrows[e, pl.ds(16 * v, 16)]
                    msgbuf[e, pl.ds(16 * v, 16)] = cexp * hv
            pltpu.sync_copy(msgbuf, acc.at[dstbuf.at[j]], add=True)
            return c
        lax.fori_loop(0, ch, chunk, 0)
        plsc.subcore_barrier()

        pltpu.sync_copy(acc.at[pl.ds(sid * rows_per_tile, rows_per_tile)],
                        op_hbm.at[cid, pl.ds(sid * rows_per_tile,
                                             rows_per_tile)])

    return body


def _block_diag_att(att, dup):
    heads, hd = att.shape
    dh = heads * hd
    rows = jnp.arange(dh)
    cols = jnp.repeat(jnp.arange(heads), hd)
    m = jnp.zeros((dh, heads), F32).at[rows, cols].set(att.reshape(dh))
    if heads == 1:
        m = jnp.tile(m, (1, 8))
    if dup:
        m = jnp.concatenate([m, m], axis=1)
    return m


def _gat_layer(src3, dst3, n, ch, ep_real, x, W, att_src, att_dst,
               heads, mid, bias_in=None):
    asb = _block_diag_att(att_src, dup=True)
    adb = _block_diag_att(att_dst, dup=False)
    wT = W.T
    if mid:
        hs, s, ad = _tc_prep(x, wT, asb, adb, mid=True,
                             b=bias_in.reshape(1, -1))
    else:
        hs, s, ad = _tc_prep(x, wT, asb, adb, mid=False)
    mp = _sc_phase_a(n, ch, ep_real)(src3, dst3, s)
    r = _tc_amax(mp.reshape(NT, n, 8), ad)
    ea, dp = _sc_phase_b(n, ch, ep_real)(src3, dst3, s, r)
    t = _tc_denom(dp.reshape(NT, n, 8))
    phase_c = _sc_phase_c(n, ch, 64, heads)
    return [phase_c(src3, dst3, t, ea, hp) for hp in hs]


def kernel(x, edge_index, W1, att_src1, att_dst1, b1,
           W2, att_src2, att_dst2, b2):
    n = x.shape[0]
    e = edge_index.shape[1]
    ep_real = e + n
    ch = -(-ep_real // (NT * K))
    epp = NT * ch * K
    loops = jnp.arange(n, dtype=edge_index.dtype)
    pad = jnp.zeros((epp - ep_real,), edge_index.dtype)
    src1d = jnp.concatenate([edge_index[0], loops, pad])
    dst1d = jnp.concatenate([edge_index[1], loops, pad])
    src3 = src1d.reshape(NT, ch, K)
    dst3 = dst1d.reshape(NT, ch, K)

    op1 = _gat_layer(src3, dst3, n, ch, ep_real, x, W1, att_src1,
                     att_dst1, heads=8, mid=False)
    op2 = _gat_layer(src3, dst3, n, ch, ep_real, op1[0], W2, att_src2,
                     att_dst2, heads=1, mid=True, bias_in=b1)
    return _tc_fin(op2[0], op2[1], b2.reshape(1, -1))


# speedup vs baseline: 21.6759x; 1.0080x over previous
import functools

import jax
import jax.numpy as jnp
from jax import lax
from jax.experimental import pallas as pl
from jax.experimental.pallas import tpu as pltpu
from jax.experimental.pallas import tpu_sc as plsc

NT = 32
NSUB = 16
K = 128
BN = 400
F32 = jnp.float32


def _splat(v, dtype=jnp.int32):
    return jnp.full((16,), v, dtype)


def _dyn_gather(x, idx):
    return lax.gather(
        x, idx[:, None],
        lax.GatherDimensionNumbers(offset_dims=(), collapsed_slice_dims=(0,),
                                   start_index_map=(0,)),
        (1,), mode=lax.GatherScatterMode.PROMISE_IN_BOUNDS)


def _dot(a, b):
    return jnp.dot(a, b, preferred_element_type=F32)


def _make_prep_kernel(nparts, mid):
    def kern(*refs):
        if mid:
            p_ref, b_ref, w_ref, asb_ref, adb_ref = refs[:5]
            outs = refs[5:]
            o = p_ref[0] + p_ref[1] + b_ref[...]
        else:
            o_ref, w_ref, asb_ref, adb_ref = refs[:4]
            outs = refs[4:]
            o = o_ref[...]
        h = _dot(o, w_ref[...])
        for i in range(nparts):
            outs[i][...] = h[:, 64 * i:64 * (i + 1)]
        outs[nparts][...] = _dot(h, asb_ref[...])
        outs[nparts + 1][...] = _dot(h, adb_ref[...])
    return kern


def _amax_kernel(mp_ref, ad_ref, r_ref):
    m = jnp.max(mp_ref[...], axis=0)
    ad = ad_ref[...]
    am = ad + m
    am = jnp.where(am >= 0, am, 0.2 * am)
    r_ref[...] = jnp.concatenate([ad, am], axis=1)


def _denom_kernel(dp_ref, t_ref):
    t = 1.0 / (jnp.sum(dp_ref[...], axis=0) + 1e-16)
    t_ref[...] = jnp.concatenate([t, t], axis=1)


def _fin_kernel(pa_ref, pb_ref, b_ref, o_ref):
    o_ref[...] = jnp.concatenate([pa_ref[0] + pa_ref[1],
                                  pb_ref[0] + pb_ref[1]], axis=1) + b_ref[...]


def _tc_prep(x, w, asb, adb, mid, b=None):
    din, dh = w.shape
    nparts = dh // 64
    n = x.shape[1] if mid else x.shape[0]
    grid = (n // BN,)
    kern = _make_prep_kernel(nparts, mid)
    ins = [x]
    in_specs = [pl.BlockSpec(((2, BN, din) if mid else (BN, din)),
                             ((lambda i: (0, i, 0)) if mid else (lambda i: (i, 0))))]
    if mid:
        ins.append(b)
        in_specs.append(pl.BlockSpec((1, din), lambda i: (0, 0)))
    ins += [w, asb, adb]
    in_specs += [pl.BlockSpec((din, dh), lambda i: (0, 0)),
                 pl.BlockSpec((dh, 16), lambda i: (0, 0)),
                 pl.BlockSpec((dh, 8), lambda i: (0, 0))]
    out = pl.pallas_call(
        kern,
        grid=grid,
        in_specs=in_specs,
        out_specs=[pl.BlockSpec((BN, 64), lambda i: (i, 0))] * nparts +
                  [pl.BlockSpec((BN, 16), lambda i: (i, 0)),
                   pl.BlockSpec((BN, 8), lambda i: (i, 0))],
        out_shape=[jax.ShapeDtypeStruct((n, 64), F32)] * nparts +
                  [jax.ShapeDtypeStruct((n, 16), F32),
                   jax.ShapeDtypeStruct((n, 8), F32)],
    )(*ins)
    return out[:nparts], out[nparts], out[nparts + 1]


def _tc_amax(mp, ad):
    n = ad.shape[0]
    return pl.pallas_call(
        _amax_kernel,
        grid=(n // BN,),
        in_specs=[pl.BlockSpec((NT, BN, 8), lambda i: (0, i, 0)),
                  pl.BlockSpec((BN, 8), lambda i: (i, 0))],
        out_specs=pl.BlockSpec((BN, 16), lambda i: (i, 0)),
        out_shape=jax.ShapeDtypeStruct((n, 16), F32),
    )(mp, ad)


def _tc_denom(dp):
    n = dp.shape[1]
    return pl.pallas_call(
        _denom_kernel,
        grid=(n // BN,),
        in_specs=[pl.BlockSpec((NT, BN, 8), lambda i: (0, i, 0))],
        out_specs=pl.BlockSpec((BN, 16), lambda i: (i, 0)),
        out_shape=jax.ShapeDtypeStruct((n, 16), F32),
    )(dp)


def _tc_fin(pa, pb, b):
    _, n, dp = pa.shape
    d = 2 * dp
    return pl.pallas_call(
        _fin_kernel,
        grid=(n // BN,),
        in_specs=[pl.BlockSpec((2, BN, dp), lambda i: (0, i, 0)),
                  pl.BlockSpec((2, BN, dp), lambda i: (0, i, 0)),
                  pl.BlockSpec((1, d), lambda i: (0, 0))],
        out_specs=pl.BlockSpec((BN, d), lambda i: (i, 0)),
        out_shape=jax.ShapeDtypeStruct((n, d), F32),
    )(pa, pb, b)


def _wid():
    return lax.axis_index("s") * 2 + lax.axis_index("c")


@functools.cache
def _sc_phase_a(n, ch, ep_real):
    mesh = plsc.VectorSubcoreMesh(core_axis_name="c", subcore_axis_name="s")

    @functools.partial(
        pl.kernel,
        out_type=jax.ShapeDtypeStruct((NT, n * 8), F32),
        mesh=mesh,
        compiler_params=pltpu.CompilerParams(needs_layout_passes=False, use_tc_tiling_on_sc=False),
        scratch_types=[
            pltpu.VMEM((ch, K), jnp.int32),
            pltpu.VMEM((ch, K), jnp.int32),
            pltpu.VMEM((K, 16), F32),
            pltpu.VMEM((n * 8,), F32),
            pltpu.SemaphoreType.DMA,
        ],
    )
    def body(src_hbm, dst_hbm, s_hbm, mp_hbm, srcbuf, dstbuf, srows, m, sem):
        w = _wid()
        iota = lax.iota(jnp.int32, 16)
        col = lax.bitwise_and(iota, 7)
        mask8 = iota < 8

        def initb(i, c):
            m[pl.ds(i * 16, 16)] = jnp.full((16,), -1e30, F32)
            return c
        lax.fori_loop(0, n * 8 // 16, initb, 0)

        pltpu.sync_copy(src_hbm.at[w], srcbuf)
        pltpu.sync_copy(dst_hbm.at[w], dstbuf)

        swapidx = lax.bitwise_xor(iota, 8)

        def chunk(j, c):
            pltpu.async_copy(s_hbm.at[srcbuf.at[j]], srows, sem).wait()

            def edge(q, c2):
                e0 = 2 * q
                e1 = 2 * q + 1
                comb = jnp.where(mask8, srows[e0], srows[e1])
                dv0 = plsc.load_gather(dstbuf, [_splat(j), _splat(e0)])
                dv1 = plsc.load_gather(dstbuf, [_splat(j), _splat(e1)])
                midx = jnp.where(mask8, dv0, dv1) * 8 + col
                eqd = dv0 == dv1
                swapped = _dyn_gather(comb, swapidx)
                val = jnp.where(eqd, jnp.maximum(comb, swapped), comb)
                mk = mask8 | (dv0 != dv1)
                cur = plsc.load_gather(m, [midx], mask=mk)
                plsc.store_scatter(m, [midx], jnp.maximum(cur, val), mask=mk)
                return c2
            lax.fori_loop(0, K // 2, edge, 0)
            return c
        lax.fori_loop(0, ch, chunk, 0)
        pltpu.sync_copy(m, mp_hbm.at[w])

    return body


@functools.cache
def _sc_phase_b(n, ch, ep_real):
    mesh = plsc.VectorSubcoreMesh(core_axis_name="c", subcore_axis_name="s")

    @functools.partial(
        pl.kernel,
        out_type=[jax.ShapeDtypeStruct((NT * ch, K * 8), F32),
                  jax.ShapeDtypeStruct((NT, n * 8), F32)],
        mesh=mesh,
        compiler_params=pltpu.CompilerParams(needs_layout_passes=False, use_tc_tiling_on_sc=False),
        scratch_types=[
            pltpu.VMEM((ch, K), jnp.int32),
            pltpu.VMEM((ch, K), jnp.int32),
            pltpu.VMEM((K, 16), F32),
            pltpu.VMEM((K, 16), F32),
            pltpu.VMEM((K * 8,), F32),
            pltpu.VMEM((n * 8,), F32),
            pltpu.SemaphoreType.DMA,
            pltpu.SemaphoreType.DMA,
        ],
    )
    def body(src_hbm, dst_hbm, s_hbm, r_hbm, ea_hbm, dp_hbm,
             srcbuf, dstbuf, srows, rrows, eabuf, dnm, sem, sem2):
        w = _wid()
        iota = lax.iota(jnp.int32, 16)
        col = lax.bitwise_and(iota, 7)
        mask8 = iota < 8

        def initb(i, c):
            dnm[pl.ds(i * 16, 16)] = jnp.zeros((16,), F32)
            return c
        lax.fori_loop(0, n * 8 // 16, initb, 0)

        pltpu.sync_copy(src_hbm.at[w], srcbuf)
        pltpu.sync_copy(dst_hbm.at[w], dstbuf)

        swapidx = lax.bitwise_xor(iota, 8)

        def chunk(j, c):
            pltpu.async_copy(s_hbm.at[srcbuf.at[j]], srows, sem).wait()
            pltpu.async_copy(r_hbm.at[dstbuf.at[j]], rrows, sem2).wait()
            base = (w * ch + j) * K

            @plsc.parallel_loop(0, K // 2)
            def edge(q):
                e0 = 2 * q
                e1 = 2 * q + 1
                sv = jnp.where(mask8, srows[e0], srows[e1])
                rv0 = rrows[e0]
                rv1 = rrows[e1]
                rlo = jnp.where(mask8, rv0, _dyn_gather(rv1, swapidx))
                rhi = jnp.where(mask8, _dyn_gather(rv0, swapidx), rv1)
                alpha = sv + rlo
                alpha = jnp.where(alpha >= 0, alpha, 0.2 * alpha)
                ea = jnp.exp(alpha - rhi)
                gid = jnp.where(mask8, _splat(base + e0), _splat(base + e1))
                ea = jnp.where(gid < ep_real, ea, jnp.zeros((16,), F32))
                eabuf[pl.ds(q * 16, 16)] = ea
                dv0 = plsc.load_gather(dstbuf, [_splat(j), _splat(e0)])
                dv1 = plsc.load_gather(dstbuf, [_splat(j), _splat(e1)])
                didx = jnp.where(mask8, dv0, dv1) * 8 + col
                eqd = dv0 == dv1
                val = jnp.where(eqd, ea + _dyn_gather(ea, swapidx), ea)
                mk = mask8 | (dv0 != dv1)
                plsc.addupdate_scatter(dnm, [didx], val, mask=mk)
            pltpu.sync_copy(eabuf, ea_hbm.at[w * ch + j])
            return c
        lax.fori_loop(0, ch, chunk, 0)
        pltpu.sync_copy(dnm, dp_hbm.at[w])

    return body


@functools.cache
def _sc_phase_c(n, ch, d, heads):
    nv = d // 16
    rows_per_tile = n // NSUB
    zrows = 5
    mesh = plsc.VectorSubcoreMesh(core_axis_name="c", subcore_axis_name="s")

    @functools.partial(
        pl.kernel,
        out_type=jax.ShapeDtypeStruct((2, n, d), F32),
        mesh=mesh,
        compiler_params=pltpu.CompilerParams(needs_layout_passes=False, use_tc_tiling_on_sc=False),
        scratch_types=[
            pltpu.VMEM((ch, K), jnp.int32),
            pltpu.VMEM((ch, K), jnp.int32),
            pltpu.VMEM((K, 16), F32),
            pltpu.VMEM((K * 8,), F32),
            pltpu.VMEM((K, d), F32),
            pltpu.VMEM((K, d), F32),
            pltpu.VMEM((zrows, d), F32),
            pltpu.VMEM_SHARED((n, d), F32),
            pltpu.SemaphoreType.DMA,
            pltpu.SemaphoreType.DMA,
        ],
    )
    def body(src_hbm, dst_hbm, t_hbm, ea_hbm, h_hbm, op_hbm,
             srcbuf, dstbuf, trows, eabuf, hrows, msgbuf, zbuf, acc, sem, sem2):
        cid = lax.axis_index("c")
        sid = lax.axis_index("s")
        w = _wid()
        iota = lax.iota(jnp.int32, 16)
        col = lax.bitwise_and(iota, 7)

        zero = jnp.zeros((16,), F32)
        for zi in range(zrows):
            for zv in range(nv):
                zbuf[zi, pl.ds(16 * zv, 16)] = zero

        def zinit(i, c):
            pltpu.sync_copy(zbuf, acc.at[pl.ds(sid * rows_per_tile + i * zrows,
                                               zrows)])
            return c
        lax.fori_loop(0, rows_per_tile // zrows, zinit, 0)
        plsc.subcore_barrier()

        pltpu.sync_copy(src_hbm.at[w], srcbuf)
        pltpu.sync_copy(dst_hbm.at[w], dstbuf)

        if heads == 1:
            pats = [jnp.zeros((16,), jnp.int32)] * nv
        else:
            pats = [2 * v + (iota >= 8).astype(jnp.int32) for v in range(nv)]

        def chunk(j, c):
            pltpu.async_copy(t_hbm.at[dstbuf.at[j]], trows, sem).wait()
            pltpu.async_copy(h_hbm.at[srcbuf.at[j]], hrows, sem2).wait()
            pltpu.sync_copy(ea_hbm.at[w * ch + j], eabuf)

            @plsc.parallel_loop(0, K)
            def edge(e):
                ea16 = plsc.load_gather(eabuf, [e * 8 + col])
                coef = ea16 * trows[e]
                for v in range(nv):
                    cexp = _dyn_gather(coef, pats[v])
                    hv = hrows[e, pl.ds(16 * v, 16)]
                    msgbuf[e, pl.ds(16 * v, 16)] = cexp * hv
            pltpu.sync_copy(msgbuf, acc.at[dstbuf.at[j]], add=True)
            return c
        lax.fori_loop(0, ch, chunk, 0)
        plsc.subcore_barrier()

        pltpu.sync_copy(acc.at[pl.ds(sid * rows_per_tile, rows_per_tile)],
                        op_hbm.at[cid, pl.ds(sid * rows_per_tile,
                                             rows_per_tile)])

    return body


def _block_diag_att(att, dup):
    heads, hd = att.shape
    dh = heads * hd
    rows = jnp.arange(dh)
    cols = jnp.repeat(jnp.arange(heads), hd)
    m = jnp.zeros((dh, heads), F32).at[rows, cols].set(att.reshape(dh))
    if heads == 1:
        m = jnp.tile(m, (1, 8))
    if dup:
        m = jnp.concatenate([m, m], axis=1)
    return m


def _gat_layer(src3, dst3, n, ch, ep_real, x, W, att_src, att_dst,
               heads, mid, bias_in=None):
    asb = _block_diag_att(att_src, dup=True)
    adb = _block_diag_att(att_dst, dup=False)
    wT = W.T
    if mid:
        hs, s, ad = _tc_prep(x, wT, asb, adb, mid=True,
                             b=bias_in.reshape(1, -1))
    else:
        hs, s, ad = _tc_prep(x, wT, asb, adb, mid=False)
    mp = _sc_phase_a(n, ch, ep_real)(src3, dst3, s)
    r = _tc_amax(mp.reshape(NT, n, 8), ad)
    ea, dp = _sc_phase_b(n, ch, ep_real)(src3, dst3, s, r)
    t = _tc_denom(dp.reshape(NT, n, 8))
    phase_c = _sc_phase_c(n, ch, 64, heads)
    return [phase_c(src3, dst3, t, ea, hp) for hp in hs]


def kernel(x, edge_index, W1, att_src1, att_dst1, b1,
           W2, att_src2, att_dst2, b2):
    n = x.shape[0]
    e = edge_index.shape[1]
    ep_real = e + n
    ch = -(-ep_real // (NT * K))
    epp = NT * ch * K
    loops = jnp.arange(n, dtype=edge_index.dtype)
    pad = jnp.zeros((epp - ep_real,), edge_index.dtype)
    src1d = jnp.concatenate([edge_index[0], loops, pad])
    dst1d = jnp.concatenate([edge_index[1], loops, pad])
    src3 = src1d.reshape(NT, ch, K)
    dst3 = dst1d.reshape(NT, ch, K)

    op1 = _gat_layer(src3, dst3, n, ch, ep_real, x, W1, att_src1,
                     att_dst1, heads=8, mid=False)
    op2 = _gat_layer(src3, dst3, n, ch, ep_real, op1[0], W2, att_src2,
                     att_dst2, heads=1, mid=True, bias_in=b1)
    return _tc_fin(op2[0], op2[1], b2.reshape(1, -1))


# speedup vs baseline: 21.8179x; 1.0066x over previous
import functools

import jax
import jax.numpy as jnp
from jax import lax
from jax.experimental import pallas as pl
from jax.experimental.pallas import tpu as pltpu
from jax.experimental.pallas import tpu_sc as plsc

NT = 32
NSUB = 16
K = 128
BN = 400
F32 = jnp.float32


def _splat(v, dtype=jnp.int32):
    return jnp.full((16,), v, dtype)


def _dyn_gather(x, idx):
    return lax.gather(
        x, idx[:, None],
        lax.GatherDimensionNumbers(offset_dims=(), collapsed_slice_dims=(0,),
                                   start_index_map=(0,)),
        (1,), mode=lax.GatherScatterMode.PROMISE_IN_BOUNDS)


def _dot(a, b):
    return jnp.dot(a, b, preferred_element_type=F32)


def _make_prep_kernel(nparts, mid):
    def kern(*refs):
        if mid:
            p_ref, b_ref, w_ref, asb_ref, adb_ref = refs[:5]
            outs = refs[5:]
            o = p_ref[0] + p_ref[1] + b_ref[...]
        else:
            o_ref, w_ref, asb_ref, adb_ref = refs[:4]
            outs = refs[4:]
            o = o_ref[...]
        h = _dot(o, w_ref[...])
        for i in range(nparts):
            outs[i][...] = h[:, 64 * i:64 * (i + 1)]
        outs[nparts][...] = _dot(h, asb_ref[...])
        outs[nparts + 1][...] = _dot(h, adb_ref[...])
    return kern


def _amax_kernel(mp_ref, ad_ref, r_ref):
    m = jnp.max(mp_ref[...], axis=0)
    ad = ad_ref[...]
    am = ad + m
    am = jnp.where(am >= 0, am, 0.2 * am)
    r_ref[...] = jnp.concatenate([ad, am], axis=1)


def _denom_kernel(dp_ref, t_ref):
    t = 1.0 / (jnp.sum(dp_ref[...], axis=0) + 1e-16)
    t_ref[...] = jnp.concatenate([t, t], axis=1)


def _fin_kernel(pa_ref, pb_ref, b_ref, o_ref):
    o_ref[...] = jnp.concatenate([pa_ref[0] + pa_ref[1],
                                  pb_ref[0] + pb_ref[1]], axis=1) + b_ref[...]


def _tc_prep(x, w, asb, adb, mid, b=None):
    din, dh = w.shape
    nparts = dh // 64
    n = x.shape[1] if mid else x.shape[0]
    grid = (n // BN,)
    kern = _make_prep_kernel(nparts, mid)
    ins = [x]
    in_specs = [pl.BlockSpec(((2, BN, din) if mid else (BN, din)),
                             ((lambda i: (0, i, 0)) if mid else (lambda i: (i, 0))))]
    if mid:
        ins.append(b)
        in_specs.append(pl.BlockSpec((1, din), lambda i: (0, 0)))
    ins += [w, asb, adb]
    in_specs += [pl.BlockSpec((din, dh), lambda i: (0, 0)),
                 pl.BlockSpec((dh, 16), lambda i: (0, 0)),
                 pl.BlockSpec((dh, 8), lambda i: (0, 0))]
    out = pl.pallas_call(
        kern,
        grid=grid,
        in_specs=in_specs,
        out_specs=[pl.BlockSpec((BN, 64), lambda i: (i, 0))] * nparts +
                  [pl.BlockSpec((BN, 16), lambda i: (i, 0)),
                   pl.BlockSpec((BN, 8), lambda i: (i, 0))],
        out_shape=[jax.ShapeDtypeStruct((n, 64), F32)] * nparts +
                  [jax.ShapeDtypeStruct((n, 16), F32),
                   jax.ShapeDtypeStruct((n, 8), F32)],
    )(*ins)
    return out[:nparts], out[nparts], out[nparts + 1]


def _tc_amax(mp, ad):
    n = ad.shape[0]
    return pl.pallas_call(
        _amax_kernel,
        grid=(n // BN,),
        in_specs=[pl.BlockSpec((NT, BN, 8), lambda i: (0, i, 0)),
                  pl.BlockSpec((BN, 8), lambda i: (i, 0))],
        out_specs=pl.BlockSpec((BN, 16), lambda i: (i, 0)),
        out_shape=jax.ShapeDtypeStruct((n, 16), F32),
    )(mp, ad)


def _tc_denom(dp):
    n = dp.shape[1]
    return pl.pallas_call(
        _denom_kernel,
        grid=(n // BN,),
        in_specs=[pl.BlockSpec((NT, BN, 8), lambda i: (0, i, 0))],
        out_specs=pl.BlockSpec((BN, 16), lambda i: (i, 0)),
        out_shape=jax.ShapeDtypeStruct((n, 16), F32),
    )(dp)


def _tc_fin(pa, pb, b):
    _, n, dp = pa.shape
    d = 2 * dp
    return pl.pallas_call(
        _fin_kernel,
        grid=(n // BN,),
        in_specs=[pl.BlockSpec((2, BN, dp), lambda i: (0, i, 0)),
                  pl.BlockSpec((2, BN, dp), lambda i: (0, i, 0)),
                  pl.BlockSpec((1, d), lambda i: (0, 0))],
        out_specs=pl.BlockSpec((BN, d), lambda i: (i, 0)),
        out_shape=jax.ShapeDtypeStruct((n, d), F32),
    )(pa, pb, b)


def _wid():
    return lax.axis_index("s") * 2 + lax.axis_index("c")


@functools.cache
def _sc_phase_a(n, ch, ep_real):
    mesh = plsc.VectorSubcoreMesh(core_axis_name="c", subcore_axis_name="s")

    @functools.partial(
        pl.kernel,
        out_type=jax.ShapeDtypeStruct((NT, n * 8), F32),
        mesh=mesh,
        compiler_params=pltpu.CompilerParams(needs_layout_passes=False, use_tc_tiling_on_sc=False),
        scratch_types=[
            pltpu.VMEM((ch, K), jnp.int32),
            pltpu.VMEM((ch, K), jnp.int32),
            pltpu.VMEM((K, 16), F32),
            pltpu.VMEM((n * 8,), F32),
            pltpu.SemaphoreType.DMA,
        ],
    )
    def body(src_hbm, dst_hbm, s_hbm, mp_hbm, srcbuf, dstbuf, srows, m, sem):
        w = _wid()
        iota = lax.iota(jnp.int32, 16)
        col = lax.bitwise_and(iota, 7)
        mask8 = iota < 8

        def initb(i, c):
            m[pl.ds(i * 16, 16)] = jnp.full((16,), -1e30, F32)
            return c
        lax.fori_loop(0, n * 8 // 16, initb, 0)

        pltpu.sync_copy(src_hbm.at[w], srcbuf)
        pltpu.sync_copy(dst_hbm.at[w], dstbuf)

        swapidx = lax.bitwise_xor(iota, 8)

        def chunk(j, c):
            pltpu.async_copy(s_hbm.at[srcbuf.at[j]], srows, sem).wait()

            def edge(q, c2):
                e0 = 2 * q
                e1 = 2 * q + 1
                comb = jnp.where(mask8, srows[e0], srows[e1])
                dv0 = plsc.load_gather(dstbuf, [_splat(j), _splat(e0)])
                dv1 = plsc.load_gather(dstbuf, [_splat(j), _splat(e1)])
                midx = jnp.where(mask8, dv0, dv1) * 8 + col
                eqd = dv0 == dv1
                swapped = _dyn_gather(comb, swapidx)
                val = jnp.where(eqd, jnp.maximum(comb, swapped), comb)
                mk = mask8 | (dv0 != dv1)
                cur = plsc.load_gather(m, [midx], mask=mk)
                plsc.store_scatter(m, [midx], jnp.maximum(cur, val), mask=mk)
                return c2
            lax.fori_loop(0, K // 2, edge, 0)
            return c
        lax.fori_loop(0, ch, chunk, 0)
        pltpu.sync_copy(m, mp_hbm.at[w])

    return body


@functools.cache
def _sc_phase_b(n, ch, ep_real):
    mesh = plsc.VectorSubcoreMesh(core_axis_name="c", subcore_axis_name="s")

    @functools.partial(
        pl.kernel,
        out_type=[jax.ShapeDtypeStruct((NT * ch, K * 8), F32),
                  jax.ShapeDtypeStruct((NT, n * 8), F32)],
        mesh=mesh,
        compiler_params=pltpu.CompilerParams(needs_layout_passes=False, use_tc_tiling_on_sc=False),
        scratch_types=[
            pltpu.VMEM((ch, K), jnp.int32),
            pltpu.VMEM((ch, K), jnp.int32),
            pltpu.VMEM((K, 16), F32),
            pltpu.VMEM((K, 16), F32),
            pltpu.VMEM((K * 8,), F32),
            pltpu.VMEM((n * 8,), F32),
            pltpu.SemaphoreType.DMA,
            pltpu.SemaphoreType.DMA,
        ],
    )
    def body(src_hbm, dst_hbm, s_hbm, r_hbm, ea_hbm, dp_hbm,
             srcbuf, dstbuf, srows, rrows, eabuf, dnm, sem, sem2):
        w = _wid()
        iota = lax.iota(jnp.int32, 16)
        col = lax.bitwise_and(iota, 7)
        mask8 = iota < 8

        def initb(i, c):
            dnm[pl.ds(i * 16, 16)] = jnp.zeros((16,), F32)
            return c
        lax.fori_loop(0, n * 8 // 16, initb, 0)

        pltpu.sync_copy(src_hbm.at[w], srcbuf)
        pltpu.sync_copy(dst_hbm.at[w], dstbuf)

        swapidx = lax.bitwise_xor(iota, 8)

        def chunk(j, c):
            pltpu.async_copy(s_hbm.at[srcbuf.at[j]], srows, sem).wait()
            pltpu.async_copy(r_hbm.at[dstbuf.at[j]], rrows, sem2).wait()
            base = (w * ch + j) * K

            @plsc.parallel_loop(0, K // 2, unroll=2)
            def edge(q):
                e0 = 2 * q
                e1 = 2 * q + 1
                sv = jnp.where(mask8, srows[e0], srows[e1])
                rv0 = rrows[e0]
                rv1 = rrows[e1]
                rlo = jnp.where(mask8, rv0, _dyn_gather(rv1, swapidx))
                rhi = jnp.where(mask8, _dyn_gather(rv0, swapidx), rv1)
                alpha = sv + rlo
                alpha = jnp.where(alpha >= 0, alpha, 0.2 * alpha)
                ea = jnp.exp(alpha - rhi)
                gid = jnp.where(mask8, _splat(base + e0), _splat(base + e1))
                ea = jnp.where(gid < ep_real, ea, jnp.zeros((16,), F32))
                eabuf[pl.ds(q * 16, 16)] = ea
                dv0 = plsc.load_gather(dstbuf, [_splat(j), _splat(e0)])
                dv1 = plsc.load_gather(dstbuf, [_splat(j), _splat(e1)])
                didx = jnp.where(mask8, dv0, dv1) * 8 + col
                eqd = dv0 == dv1
                val = jnp.where(eqd, ea + _dyn_gather(ea, swapidx), ea)
                mk = mask8 | (dv0 != dv1)
                plsc.addupdate_scatter(dnm, [didx], val, mask=mk)
            pltpu.sync_copy(eabuf, ea_hbm.at[w * ch + j])
            return c
        lax.fori_loop(0, ch, chunk, 0)
        pltpu.sync_copy(dnm, dp_hbm.at[w])

    return body


@functools.cache
def _sc_phase_c(n, ch, d, heads):
    nv = d // 16
    rows_per_tile = n // NSUB
    zrows = 5
    mesh = plsc.VectorSubcoreMesh(core_axis_name="c", subcore_axis_name="s")

    @functools.partial(
        pl.kernel,
        out_type=jax.ShapeDtypeStruct((2, n, d), F32),
        mesh=mesh,
        compiler_params=pltpu.CompilerParams(needs_layout_passes=False, use_tc_tiling_on_sc=False),
        scratch_types=[
            pltpu.VMEM((ch, K), jnp.int32),
            pltpu.VMEM((ch, K), jnp.int32),
            pltpu.VMEM((K, 16), F32),
            pltpu.VMEM((K * 8,), F32),
            pltpu.VMEM((K, d), F32),
            pltpu.VMEM((K, d), F32),
            pltpu.VMEM((zrows, d), F32),
            pltpu.VMEM_SHARED((n, d), F32),
            pltpu.SemaphoreType.DMA,
            pltpu.SemaphoreType.DMA,
        ],
    )
    def body(src_hbm, dst_hbm, t_hbm, ea_hbm, h_hbm, op_hbm,
             srcbuf, dstbuf, trows, eabuf, hrows, msgbuf, zbuf, acc, sem, sem2):
        cid = lax.axis_index("c")
        sid = lax.axis_index("s")
        w = _wid()
        iota = lax.iota(jnp.int32, 16)
        col = lax.bitwise_and(iota, 7)

        zero = jnp.zeros((16,), F32)
        for zi in range(zrows):
            for zv in range(nv):
                zbuf[zi, pl.ds(16 * zv, 16)] = zero

        def zinit(i, c):
            pltpu.sync_copy(zbuf, acc.at[pl.ds(sid * rows_per_tile + i * zrows,
                                               zrows)])
            return c
        lax.fori_loop(0, rows_per_tile // zrows, zinit, 0)
        plsc.subcore_barrier()

        pltpu.sync_copy(src_hbm.at[w], srcbuf)
        pltpu.sync_copy(dst_hbm.at[w], dstbuf)

        if heads == 1:
            pats = [jnp.zeros((16,), jnp.int32)] * nv
        else:
            pats = [2 * v + (iota >= 8).astype(jnp.int32) for v in range(nv)]

        def chunk(j, c):
            pltpu.async_copy(t_hbm.at[dstbuf.at[j]], trows, sem).wait()
            pltpu.async_copy(h_hbm.at[srcbuf.at[j]], hrows, sem2).wait()
            pltpu.sync_copy(ea_hbm.at[w * ch + j], eabuf)

            @plsc.parallel_loop(0, K, unroll=2)
            def edge(e):
                ea16 = plsc.load_gather(eabuf, [e * 8 + col])
                coef = ea16 * trows[e]
                for v in range(nv):
                    cexp = _dyn_gather(coef, pats[v])
                    hv = hrows[e, pl.ds(16 * v, 16)]
                    msgbuf[e, pl.ds(16 * v, 16)] = cexp * hv
            pltpu.sync_copy(msgbuf, acc.at[dstbuf.at[j]], add=True)
            return c
        lax.fori_loop(0, ch, chunk, 0)
        plsc.subcore_barrier()

        pltpu.sync_copy(acc.at[pl.ds(sid * rows_per_tile, rows_per_tile)],
                        op_hbm.at[cid, pl.ds(sid * rows_per_tile,
                                             rows_per_tile)])

    return body


def _block_diag_att(att, dup):
    heads, hd = att.shape
    dh = heads * hd
    rows = jnp.arange(dh)
    cols = jnp.repeat(jnp.arange(heads), hd)
    m = jnp.zeros((dh, heads), F32).at[rows, cols].set(att.reshape(dh))
    if heads == 1:
        m = jnp.tile(m, (1, 8))
    if dup:
        m = jnp.concatenate([m, m], axis=1)
    return m


def _gat_layer(src3, dst3, n, ch, ep_real, x, W, att_src, att_dst,
               heads, mid, bias_in=None):
    asb = _block_diag_att(att_src, dup=True)
    adb = _block_diag_att(att_dst, dup=False)
    wT = W.T
    if mid:
        hs, s, ad = _tc_prep(x, wT, asb, adb, mid=True,
                             b=bias_in.reshape(1, -1))
    else:
        hs, s, ad = _tc_prep(x, wT, asb, adb, mid=False)
    mp = _sc_phase_a(n, ch, ep_real)(src3, dst3, s)
    r = _tc_amax(mp.reshape(NT, n, 8), ad)
    ea, dp = _sc_phase_b(n, ch, ep_real)(src3, dst3, s, r)
    t = _tc_denom(dp.reshape(NT, n, 8))
    phase_c = _sc_phase_c(n, ch, 64, heads)
    return [phase_c(src3, dst3, t, ea, hp) for hp in hs]


def kernel(x, edge_index, W1, att_src1, att_dst1, b1,
           W2, att_src2, att_dst2, b2):
    n = x.shape[0]
    e = edge_index.shape[1]
    ep_real = e + n
    ch = -(-ep_real // (NT * K))
    epp = NT * ch * K
    loops = jnp.arange(n, dtype=edge_index.dtype)
    pad = jnp.zeros((epp - ep_real,), edge_index.dtype)
    src1d = jnp.concatenate([edge_index[0], loops, pad])
    dst1d = jnp.concatenate([edge_index[1], loops, pad])
    src3 = src1d.reshape(NT, ch, K)
    dst3 = dst1d.reshape(NT, ch, K)

    op1 = _gat_layer(src3, dst3, n, ch, ep_real, x, W1, att_src1,
                     att_dst1, heads=8, mid=False)
    op2 = _gat_layer(src3, dst3, n, ch, ep_real, op1[0], W2, att_src2,
                     att_dst2, heads=1, mid=True, bias_in=b1)
    return _tc_fin(op2[0], op2[1], b2.reshape(1, -1))
